# Initial kernel scaffold; baseline (speedup 1.0000x reference)
import jax
import jax.numpy as jnp
from jax.experimental import pallas as pl


def kernel(x, edge_index, edge_attr, mask, W1, att_s1, att_d1, We1, att_e1, b1, gamma, beta, W2, att_s2, att_d2, We2, att_e2, b2, Wf1, bf1, Wf2, bf2):
    raise NotImplementedError("write your pallas kernel here")


# speedup vs baseline: 15.8411x; 15.8411x over previous
import functools

import jax
import jax.numpy as jnp
from jax import lax
from jax.experimental import pallas as pl
from jax.experimental.pallas import tpu as pltpu
from jax.experimental.pallas import tpu_sc as plsc

N = 50000
E = 800000
HID = 64

NN = 50176
ER = 1600
EC = 512
NW = 32
CH = 3200
PT = 25600
EP = 819200
CH3 = 8192

F32 = jnp.float32
I32 = jnp.int32

_mesh = plsc.VectorSubcoreMesh(core_axis_name="c", subcore_axis_name="s")


def _wid():
    return lax.axis_index("s") * 2 + lax.axis_index("c")


def _zero(ref, n):
    z = jnp.zeros((16,), F32)

    def b(i, _):
        ref[pl.ds(i * 16, 16)] = z
        return 0

    lax.fori_loop(0, n // 16, b, 0)


@functools.partial(
    pl.kernel,
    out_type=[
        jax.ShapeDtypeStruct((NW, NN), F32),
        jax.ShapeDtypeStruct((NW, NN), F32),
        jax.ShapeDtypeStruct((NW, NN), F32),
        jax.ShapeDtypeStruct((EP,), F32),
    ],
    mesh=_mesh,
    compiler_params=pltpu.CompilerParams(needs_layout_passes=False),
    scratch_types=[
        pltpu.VMEM((NN,), F32),
        pltpu.VMEM((NN,), F32),
        pltpu.VMEM((CH,), I32),
        pltpu.VMEM((CH,), I32),
        pltpu.VMEM((CH,), F32),
        pltpu.VMEM((CH,), F32),
        pltpu.VMEM((CH,), F32),
    ],
)
def _sc_m1(dstP, srcP, ae1P, ae2P, as1, degp, aes1p, aes2p, p1,
           sA, sB, ibA, ibB, fbA, fbB, fbC):
    wid = _wid()
    base = wid * PT
    ones = jnp.ones((16,), F32)

    _zero(sA, NN)
    _zero(sB, NN)

    def chunkA(i, _):
        off = base + i * CH
        pltpu.sync_copy(dstP.at[pl.ds(off, CH)], ibA)
        pltpu.sync_copy(ae1P.at[pl.ds(off, CH)], fbA)

        def b16(j, _):
            d16 = ibA[pl.ds(j * 16, 16)]
            plsc.addupdate_scatter(sA, [d16], ones)
            plsc.addupdate_scatter(sB, [d16], fbA[pl.ds(j * 16, 16)])
            return 0

        lax.fori_loop(0, CH // 16, b16, 0)
        return 0

    lax.fori_loop(0, PT // CH, chunkA, 0)
    pltpu.sync_copy(sA, degp.at[wid])
    pltpu.sync_copy(sB, aes1p.at[wid])

    pltpu.sync_copy(as1, sA)
    _zero(sB, NN)

    def chunkB(i, _):
        off = base + i * CH
        pltpu.sync_copy(srcP.at[pl.ds(off, CH)], ibA)
        pltpu.sync_copy(dstP.at[pl.ds(off, CH)], ibB)
        pltpu.sync_copy(ae1P.at[pl.ds(off, CH)], fbA)
        pltpu.sync_copy(ae2P.at[pl.ds(off, CH)], fbB)

        def b16(j, _):
            s16 = ibA[pl.ds(j * 16, 16)]
            d16 = ibB[pl.ds(j * 16, 16)]
            pv = plsc.load_gather(sA, [s16]) + fbA[pl.ds(j * 16, 16)]
            fbC[pl.ds(j * 16, 16)] = pv
            plsc.addupdate_scatter(sB, [d16], fbB[pl.ds(j * 16, 16)])
            return 0

        lax.fori_loop(0, CH // 16, b16, 0)
        pltpu.sync_copy(fbC, p1.at[pl.ds(off, CH)])
        return 0

    lax.fori_loop(0, PT // CH, chunkB, 0)
    pltpu.sync_copy(sB, aes2p.at[wid])


@functools.partial(
    pl.kernel,
    out_type=[
        jax.ShapeDtypeStruct((EP,), F32),
        jax.ShapeDtypeStruct((NW, NN), F32),
    ],
    mesh=_mesh,
    compiler_params=pltpu.CompilerParams(needs_layout_passes=False),
    scratch_types=[
        pltpu.VMEM((NN,), F32),
        pltpu.VMEM((NN,), F32),
        pltpu.VMEM((CH,), I32),
        pltpu.VMEM((CH,), F32),
        pltpu.VMEM((CH,), F32),
    ],
)
def _sc_m2(dstP, pP, ad, exO, denp, sA, sB, ibA, fbA, fbB):
    wid = _wid()
    base = wid * PT
    pltpu.sync_copy(ad, sA)
    _zero(sB, NN)

    def chunk(i, _):
        off = base + i * CH
        pltpu.sync_copy(dstP.at[pl.ds(off, CH)], ibA)
        pltpu.sync_copy(pP.at[pl.ds(off, CH)], fbA)

        def b16(j, _):
            d16 = ibA[pl.ds(j * 16, 16)]
            a = fbA[pl.ds(j * 16, 16)] + plsc.load_gather(sA, [d16])
            a = jnp.where(a > 0, a, 0.2 * a)
            ex = jnp.exp(jnp.minimum(a, 60.0))
            fbB[pl.ds(j * 16, 16)] = ex
            plsc.addupdate_scatter(sB, [d16], ex)
            return 0

        lax.fori_loop(0, CH // 16, b16, 0)
        pltpu.sync_copy(fbB, exO.at[pl.ds(off, CH)])
        return 0

    lax.fori_loop(0, PT // CH, chunk, 0)
    pltpu.sync_copy(sB, denp.at[wid])


@functools.partial(
    pl.kernel,
    out_type=[
        jax.ShapeDtypeStruct((EP,), F32),
        jax.ShapeDtypeStruct((EP,), F32),
        jax.ShapeDtypeStruct((NW, NN), F32),
    ],
    mesh=_mesh,
    compiler_params=pltpu.CompilerParams(needs_layout_passes=False),
    scratch_types=[
        pltpu.VMEM((NN,), F32),
        pltpu.VMEM((NN,), F32),
        pltpu.VMEM((CH,), I32),
        pltpu.VMEM((CH,), F32),
        pltpu.VMEM((CH,), F32),
    ],
)
def _sc_m4(srcP, dstP, ae2P, as2, ad2, p2O, ex2O, denp, sA, sB, ibA, fbA, fbB):
    wid = _wid()
    base = wid * PT
    pltpu.sync_copy(as2, sA)

    def chunkA(i, _):
        off = base + i * CH
        pltpu.sync_copy(srcP.at[pl.ds(off, CH)], ibA)
        pltpu.sync_copy(ae2P.at[pl.ds(off, CH)], fbA)

        def b16(j, _):
            s16 = ibA[pl.ds(j * 16, 16)]
            fbB[pl.ds(j * 16, 16)] = (
                plsc.load_gather(sA, [s16]) + fbA[pl.ds(j * 16, 16)])
            return 0

        lax.fori_loop(0, CH // 16, b16, 0)
        pltpu.sync_copy(fbB, p2O.at[pl.ds(off, CH)])
        return 0

    lax.fori_loop(0, PT // CH, chunkA, 0)

    pltpu.sync_copy(ad2, sA)
    _zero(sB, NN)

    def chunkB(i, _):
        off = base + i * CH
        pltpu.sync_copy(dstP.at[pl.ds(off, CH)], ibA)
        pltpu.sync_copy(p2O.at[pl.ds(off, CH)], fbA)

        def b16(j, _):
            d16 = ibA[pl.ds(j * 16, 16)]
            a = fbA[pl.ds(j * 16, 16)] + plsc.load_gather(sA, [d16])
            a = jnp.where(a > 0, a, 0.2 * a)
            ex = jnp.exp(jnp.minimum(a, 60.0))
            fbB[pl.ds(j * 16, 16)] = ex
            plsc.addupdate_scatter(sB, [d16], ex)
            return 0

        lax.fori_loop(0, CH // 16, b16, 0)
        pltpu.sync_copy(fbB, ex2O.at[pl.ds(off, CH)])
        return 0

    lax.fori_loop(0, PT // CH, chunkB, 0)
    pltpu.sync_copy(sB, denp.at[wid])


@functools.partial(
    pl.kernel,
    out_type=[jax.ShapeDtypeStruct((HID, NN), F32)],
    mesh=_mesh,
    compiler_params=pltpu.CompilerParams(needs_layout_passes=False),
    scratch_types=[
        pltpu.VMEM((NN,), F32),
        pltpu.VMEM((NN,), F32),
        pltpu.VMEM((CH3,), I32),
        pltpu.VMEM((CH3,), I32),
        pltpu.VMEM((CH3,), F32),
    ],
)
def _sc_m3(srcP, dstP, exP, hT, numT, sA, sB, ibS, ibD, fbE):
    wid = _wid()
    for fi in range(2):
        f = wid + NW * fi
        pltpu.sync_copy(hT.at[f], sA)
        _zero(sB, NN)

        def chunk(i, _):
            off = i * CH3
            pltpu.sync_copy(srcP.at[pl.ds(off, CH3)], ibS)
            pltpu.sync_copy(dstP.at[pl.ds(off, CH3)], ibD)
            pltpu.sync_copy(exP.at[pl.ds(off, CH3)], fbE)

            def b16(j, _):
                s16 = ibS[pl.ds(j * 16, 16)]
                d16 = ibD[pl.ds(j * 16, 16)]
                v = plsc.load_gather(sA, [s16]) * fbE[pl.ds(j * 16, 16)]
                plsc.addupdate_scatter(sB, [d16], v)
                return 0

            lax.fori_loop(0, CH3 // 16, b16, 0)
            return 0

        lax.fori_loop(0, EP // CH3, chunk, 0)
        pltpu.sync_copy(sB, numT.at[f])


def _tc_pre_body(xT, ea0, ea1, W1T, atts1, attd1, vv,
                 h1T, as1, ad1, ae1, ae2):
    xv = xT[...]
    h = W1T[:, 0:1] * xv[0:1, :] + W1T[:, 1:2] * xv[1:2, :]
    h1T[...] = h
    as1[...] = jnp.sum(h * atts1[...], axis=0, keepdims=True)
    ad1[...] = jnp.sum(h * attd1[...], axis=0, keepdims=True)
    v = vv[...]
    e0 = ea0[...]
    e1 = ea1[...]
    ae1[...] = e0 * v[0:1, 0:1] + e1 * v[0:1, 1:2]
    ae2[...] = e0 * v[1:2, 0:1] + e1 * v[1:2, 1:2]


_tc_pre = pl.pallas_call(
    _tc_pre_body,
    out_shape=[
        jax.ShapeDtypeStruct((HID, NN), F32),
        jax.ShapeDtypeStruct((1, NN), F32),
        jax.ShapeDtypeStruct((1, NN), F32),
        jax.ShapeDtypeStruct((ER, EC), F32),
        jax.ShapeDtypeStruct((ER, EC), F32),
    ],
)


def _leaky(v, sl):
    return jnp.where(v > 0, v, sl * v)


def _tc_comb_body(degp, aes1p, aes2p, den1p, deg, aes1, aes2, den1):
    deg[...] = jnp.sum(degp[...], axis=0, keepdims=True)
    aes1[...] = jnp.sum(aes1p[...], axis=0, keepdims=True)
    aes2[...] = jnp.sum(aes2p[...], axis=0, keepdims=True)
    den1[...] = jnp.sum(den1p[...], axis=0, keepdims=True)


_tc_comb = pl.pallas_call(
    _tc_comb_body,
    out_shape=[jax.ShapeDtypeStruct((1, NN), F32)] * 4,
)


def _tc_mid_body(degR, aes1R, aes2R, den1R, as1, ad1, num1T, h1T,
                 b1, gamma, beta, W2T, atts2, attd2,
                 h2T, as2, ad2, exl2):
    deg = degR[...]
    aes1 = aes1R[...]
    aes2 = aes2R[...]
    den1 = den1R[...]
    ael1 = jnp.where(deg > 0, aes1 / jnp.maximum(deg, 1.0), 0.0)
    al1 = as1[...] + ad1[...] + ael1
    al1 = _leaky(al1, 0.2)
    exl1 = jnp.exp(jnp.minimum(al1, 60.0))
    denom1 = den1 + exl1
    gat1 = (num1T[...] + exl1 * h1T[...]) / (denom1 + 1e-16) + b1[...]
    v = gat1[:, :N]
    mu = jnp.mean(v, axis=1, keepdims=True)
    var = jnp.mean((v - mu) ** 2, axis=1, keepdims=True)
    hbn = gamma[...] * (gat1 - mu) / jnp.sqrt(var + 1e-5) + beta[...]
    hbn = _leaky(hbn, 0.01)
    h2 = jnp.dot(W2T[...], hbn, preferred_element_type=F32)
    h2T[...] = h2
    a_s = jnp.sum(h2 * atts2[...], axis=0, keepdims=True)
    a_d = jnp.sum(h2 * attd2[...], axis=0, keepdims=True)
    as2[...] = a_s
    ad2[...] = a_d
    ael2 = jnp.where(deg > 0, aes2 / jnp.maximum(deg, 1.0), 0.0)
    al2 = _leaky(a_s + a_d + ael2, 0.2)
    exl2[...] = jnp.exp(jnp.minimum(al2, 60.0))


_tc_mid = pl.pallas_call(
    _tc_mid_body,
    out_shape=[
        jax.ShapeDtypeStruct((HID, NN), F32),
        jax.ShapeDtypeStruct((1, NN), F32),
        jax.ShapeDtypeStruct((1, NN), F32),
        jax.ShapeDtypeStruct((1, NN), F32),
    ],
)


def _tc_fin_body(num2T, h2T, den2p, exl2, maskP, b2, Wf1T, bf1, Wf2T, bf2,
                 out):
    den2 = jnp.sum(den2p[...], axis=0, keepdims=True) + exl2[...]
    gat2 = (num2T[...] + exl2[...] * h2T[...]) / (den2 + 1e-16) + b2[...]
    hL = _leaky(gat2, 0.01)
    m1 = _leaky(jnp.dot(Wf1T[...], hL, preferred_element_type=F32) + bf1[...],
                0.01)
    m2 = jnp.dot(Wf2T[...], m1, preferred_element_type=F32) + bf2[...]
    out[...] = m2 * maskP[...]


_tc_fin = pl.pallas_call(
    _tc_fin_body,
    out_shape=jax.ShapeDtypeStruct((1, NN), F32),
)


def kernel(x, edge_index, edge_attr, mask, W1, att_s1, att_d1, We1, att_e1,
           b1, gamma, beta, W2, att_s2, att_d2, We2, att_e2, b2,
           Wf1, bf1, Wf2, bf2):
    src = edge_index[0].astype(I32)
    dst = edge_index[1].astype(I32)
    srcP = jnp.pad(src, (0, EP - E))
    dstP = jnp.pad(dst, (0, EP - E), constant_values=N)
    xT = jnp.pad(x.T.astype(F32), ((0, 0), (0, NN - N)))
    ea0 = jnp.pad(edge_attr[:, 0].astype(F32), (0, EP - E)).reshape(ER, EC)
    ea1 = jnp.pad(edge_attr[:, 1].astype(F32), (0, EP - E)).reshape(ER, EC)
    maskP = jnp.pad(mask.astype(F32), (0, NN - N)).reshape(1, NN)
    vv = jnp.stack([We1 @ att_e1, We2 @ att_e2]).astype(F32)

    h1T, as1, ad1, ae1, ae2 = _tc_pre(
        xT, ea0, ea1, W1.T.astype(F32),
        att_s1.reshape(HID, 1), att_d1.reshape(HID, 1), vv)

    ae1f = ae1.reshape(EP)
    ae2f = ae2.reshape(EP)
    degp, aes1p, aes2p, p1 = _sc_m1(dstP, srcP, ae1f, ae2f, as1.reshape(NN))
    ex1, den1p = _sc_m2(dstP, p1, ad1.reshape(NN))
    (num1T,) = _sc_m3(srcP, dstP, ex1, h1T)

    degC, aes1C, aes2C, den1C = _tc_comb(degp, aes1p, aes2p, den1p)
    h2T, as2, ad2, exl2 = _tc_mid(
        degC, aes1C, aes2C, den1C, as1, ad1, num1T, h1T,
        b1.reshape(HID, 1), gamma.reshape(HID, 1), beta.reshape(HID, 1),
        W2.T.astype(F32), att_s2.reshape(HID, 1), att_d2.reshape(HID, 1))

    _p2, ex2, den2p = _sc_m4(srcP, dstP, ae2f, as2.reshape(NN),
                             ad2.reshape(NN))
    (num2T,) = _sc_m3(srcP, dstP, ex2, h2T)

    out2d = _tc_fin(num2T, h2T, den2p, exl2, maskP,
                    b2.reshape(HID, 1), Wf1.T.astype(F32),
                    bf1.reshape(32, 1), Wf2.T.astype(F32),
                    bf2.reshape(1, 1))
    return out2d[0, :N]


# speedup vs baseline: 22.2703x; 1.4058x over previous
import functools

import jax
import jax.numpy as jnp
from jax import lax
from jax.experimental import pallas as pl
from jax.experimental.pallas import tpu as pltpu
from jax.experimental.pallas import tpu_sc as plsc

N = 50000
E = 800000
HID = 64

NN = 50176
ER = 1600
EC = 512
NW = 32
CH = 3200
PT = 25600
EP = 819200
CH3 = 8192

F32 = jnp.float32
I32 = jnp.int32

_mesh = plsc.VectorSubcoreMesh(core_axis_name="c", subcore_axis_name="s")


def _wid():
    return lax.axis_index("s") * 2 + lax.axis_index("c")


def _zero(ref, n):
    z = jnp.zeros((16,), F32)

    def b(i, _):
        ref[pl.ds(i * 16, 16)] = z
        return 0

    lax.fori_loop(0, n // 16, b, 0)


@functools.partial(
    pl.kernel,
    out_type=[
        jax.ShapeDtypeStruct((NW, NN), F32),
        jax.ShapeDtypeStruct((NW, NN), F32),
        jax.ShapeDtypeStruct((NW, NN), F32),
        jax.ShapeDtypeStruct((EP,), F32),
    ],
    mesh=_mesh,
    compiler_params=pltpu.CompilerParams(needs_layout_passes=False),
    scratch_types=[
        pltpu.VMEM((NN,), F32),
        pltpu.VMEM((NN,), F32),
        pltpu.VMEM((CH,), I32),
        pltpu.VMEM((CH,), I32),
        pltpu.VMEM((CH,), F32),
        pltpu.VMEM((CH,), F32),
        pltpu.VMEM((CH,), F32),
    ],
)
def _sc_m1(dstP, srcP, ae1P, ae2P, as1, degp, aes1p, aes2p, p1,
           sA, sB, ibA, ibB, fbA, fbB, fbC):
    wid = _wid()
    base = wid * PT
    ones = jnp.ones((16,), F32)

    _zero(sA, NN)
    _zero(sB, NN)

    def chunkA(i, _):
        off = base + i * CH
        pltpu.sync_copy(dstP.at[pl.ds(off, CH)], ibA)
        pltpu.sync_copy(ae1P.at[pl.ds(off, CH)], fbA)

        def b16(j, _):
            d16 = ibA[pl.ds(j * 16, 16)]
            plsc.addupdate_scatter(sA, [d16], ones)
            plsc.addupdate_scatter(sB, [d16], fbA[pl.ds(j * 16, 16)])
            return 0

        lax.fori_loop(0, CH // 16, b16, 0)
        return 0

    lax.fori_loop(0, PT // CH, chunkA, 0)
    pltpu.sync_copy(sA, degp.at[wid])
    pltpu.sync_copy(sB, aes1p.at[wid])

    pltpu.sync_copy(as1, sA)
    _zero(sB, NN)

    def chunkB(i, _):
        off = base + i * CH
        pltpu.sync_copy(srcP.at[pl.ds(off, CH)], ibA)
        pltpu.sync_copy(dstP.at[pl.ds(off, CH)], ibB)
        pltpu.sync_copy(ae1P.at[pl.ds(off, CH)], fbA)
        pltpu.sync_copy(ae2P.at[pl.ds(off, CH)], fbB)

        def b16(j, _):
            s16 = ibA[pl.ds(j * 16, 16)]
            d16 = ibB[pl.ds(j * 16, 16)]
            pv = plsc.load_gather(sA, [s16]) + fbA[pl.ds(j * 16, 16)]
            fbC[pl.ds(j * 16, 16)] = pv
            plsc.addupdate_scatter(sB, [d16], fbB[pl.ds(j * 16, 16)])
            return 0

        lax.fori_loop(0, CH // 16, b16, 0)
        pltpu.sync_copy(fbC, p1.at[pl.ds(off, CH)])
        return 0

    lax.fori_loop(0, PT // CH, chunkB, 0)
    pltpu.sync_copy(sB, aes2p.at[wid])


@functools.partial(
    pl.kernel,
    out_type=[
        jax.ShapeDtypeStruct((EP,), F32),
        jax.ShapeDtypeStruct((NW, NN), F32),
    ],
    mesh=_mesh,
    compiler_params=pltpu.CompilerParams(needs_layout_passes=False),
    scratch_types=[
        pltpu.VMEM((NN,), F32),
        pltpu.VMEM((NN,), F32),
        pltpu.VMEM((CH,), I32),
        pltpu.VMEM((CH,), F32),
        pltpu.VMEM((CH,), F32),
    ],
)
def _sc_m2(dstP, pP, ad, exO, denp, sA, sB, ibA, fbA, fbB):
    wid = _wid()
    base = wid * PT
    pltpu.sync_copy(ad, sA)
    _zero(sB, NN)

    def chunk(i, _):
        off = base + i * CH
        pltpu.sync_copy(dstP.at[pl.ds(off, CH)], ibA)
        pltpu.sync_copy(pP.at[pl.ds(off, CH)], fbA)

        def b16(j, _):
            d16 = ibA[pl.ds(j * 16, 16)]
            a = fbA[pl.ds(j * 16, 16)] + plsc.load_gather(sA, [d16])
            a = jnp.where(a > 0, a, 0.2 * a)
            ex = jnp.exp(jnp.minimum(a, 60.0))
            fbB[pl.ds(j * 16, 16)] = ex
            plsc.addupdate_scatter(sB, [d16], ex)
            return 0

        lax.fori_loop(0, CH // 16, b16, 0)
        pltpu.sync_copy(fbB, exO.at[pl.ds(off, CH)])
        return 0

    lax.fori_loop(0, PT // CH, chunk, 0)
    pltpu.sync_copy(sB, denp.at[wid])


@functools.partial(
    pl.kernel,
    out_type=[
        jax.ShapeDtypeStruct((EP,), F32),
        jax.ShapeDtypeStruct((EP,), F32),
        jax.ShapeDtypeStruct((NW, NN), F32),
    ],
    mesh=_mesh,
    compiler_params=pltpu.CompilerParams(needs_layout_passes=False),
    scratch_types=[
        pltpu.VMEM((NN,), F32),
        pltpu.VMEM((NN,), F32),
        pltpu.VMEM((CH,), I32),
        pltpu.VMEM((CH,), F32),
        pltpu.VMEM((CH,), F32),
    ],
)
def _sc_m4(srcP, dstP, ae2P, as2, ad2, p2O, ex2O, denp, sA, sB, ibA, fbA, fbB):
    wid = _wid()
    base = wid * PT
    pltpu.sync_copy(as2, sA)

    def chunkA(i, _):
        off = base + i * CH
        pltpu.sync_copy(srcP.at[pl.ds(off, CH)], ibA)
        pltpu.sync_copy(ae2P.at[pl.ds(off, CH)], fbA)

        def b16(j, _):
            s16 = ibA[pl.ds(j * 16, 16)]
            fbB[pl.ds(j * 16, 16)] = (
                plsc.load_gather(sA, [s16]) + fbA[pl.ds(j * 16, 16)])
            return 0

        lax.fori_loop(0, CH // 16, b16, 0)
        pltpu.sync_copy(fbB, p2O.at[pl.ds(off, CH)])
        return 0

    lax.fori_loop(0, PT // CH, chunkA, 0)

    pltpu.sync_copy(ad2, sA)
    _zero(sB, NN)

    def chunkB(i, _):
        off = base + i * CH
        pltpu.sync_copy(dstP.at[pl.ds(off, CH)], ibA)
        pltpu.sync_copy(p2O.at[pl.ds(off, CH)], fbA)

        def b16(j, _):
            d16 = ibA[pl.ds(j * 16, 16)]
            a = fbA[pl.ds(j * 16, 16)] + plsc.load_gather(sA, [d16])
            a = jnp.where(a > 0, a, 0.2 * a)
            ex = jnp.exp(jnp.minimum(a, 60.0))
            fbB[pl.ds(j * 16, 16)] = ex
            plsc.addupdate_scatter(sB, [d16], ex)
            return 0

        lax.fori_loop(0, CH // 16, b16, 0)
        pltpu.sync_copy(fbB, ex2O.at[pl.ds(off, CH)])
        return 0

    lax.fori_loop(0, PT // CH, chunkB, 0)
    pltpu.sync_copy(sB, denp.at[wid])


CHB = 4096
NBP = EP // CHB // 2
U3 = 4


@functools.partial(
    pl.kernel,
    out_type=[jax.ShapeDtypeStruct((HID, NN), F32)],
    mesh=_mesh,
    compiler_params=pltpu.CompilerParams(needs_layout_passes=False),
    scratch_types=[
        pltpu.VMEM((NN,), F32),
        pltpu.VMEM((NN,), F32),
        pltpu.VMEM((CHB,), I32),
        pltpu.VMEM((CHB,), I32),
        pltpu.VMEM((CHB,), F32),
        pltpu.VMEM((CHB,), I32),
        pltpu.VMEM((CHB,), I32),
        pltpu.VMEM((CHB,), F32),
        pltpu.SemaphoreType.DMA,
        pltpu.SemaphoreType.DMA,
    ],
)
def _sc_m3(srcP, dstP, exP, hT, numT,
           sA, sB, iS0, iD0, fE0, iS1, iD1, fE1, sem0, sem1):
    wid = _wid()

    def start3(c, iS, iD, fE, sem):
        off = c * CHB
        pltpu.async_copy(srcP.at[pl.ds(off, CHB)], iS, sem)
        pltpu.async_copy(dstP.at[pl.ds(off, CHB)], iD, sem)
        pltpu.async_copy(exP.at[pl.ds(off, CHB)], fE, sem)

    def wait3(c, iS, iD, fE, sem):
        off = c * CHB
        pltpu.make_async_copy(srcP.at[pl.ds(off, CHB)], iS, sem).wait()
        pltpu.make_async_copy(dstP.at[pl.ds(off, CHB)], iD, sem).wait()
        pltpu.make_async_copy(exP.at[pl.ds(off, CHB)], fE, sem).wait()

    def compute(iS, iD, fE):
        def grp(j, _):
            base = j * (16 * U3)
            for u in range(U3):
                o = base + u * 16
                s16 = iS[pl.ds(o, 16)]
                d16 = iD[pl.ds(o, 16)]
                v = plsc.load_gather(sA, [s16]) * fE[pl.ds(o, 16)]
                plsc.addupdate_scatter(sB, [d16], v)
            return 0

        lax.fori_loop(0, CHB // (16 * U3), grp, 0)

    for fi in range(2):
        f = wid + NW * fi
        pltpu.sync_copy(hT.at[f], sA)
        _zero(sB, NN)
        start3(0, iS0, iD0, fE0, sem0)

        def pair(k, _):
            c0 = 2 * k
            wait3(c0, iS0, iD0, fE0, sem0)
            start3(c0 + 1, iS1, iD1, fE1, sem1)
            compute(iS0, iD0, fE0)
            wait3(c0 + 1, iS1, iD1, fE1, sem1)

            @pl.when(k + 1 < NBP)
            def _():
                start3(c0 + 2, iS0, iD0, fE0, sem0)

            compute(iS1, iD1, fE1)
            return 0

        lax.fori_loop(0, NBP, pair, 0)
        pltpu.sync_copy(sB, numT.at[f])


def _tc_pre_body(xT, ea0, ea1, W1T, atts1, attd1, vv,
                 h1T, as1, ad1, ae1, ae2):
    xv = xT[...]
    h = W1T[:, 0:1] * xv[0:1, :] + W1T[:, 1:2] * xv[1:2, :]
    h1T[...] = h
    as1[...] = jnp.sum(h * atts1[...], axis=0, keepdims=True)
    ad1[...] = jnp.sum(h * attd1[...], axis=0, keepdims=True)
    v = vv[...]
    e0 = ea0[...]
    e1 = ea1[...]
    ae1[...] = e0 * v[0:1, 0:1] + e1 * v[0:1, 1:2]
    ae2[...] = e0 * v[1:2, 0:1] + e1 * v[1:2, 1:2]


_tc_pre = pl.pallas_call(
    _tc_pre_body,
    out_shape=[
        jax.ShapeDtypeStruct((HID, NN), F32),
        jax.ShapeDtypeStruct((1, NN), F32),
        jax.ShapeDtypeStruct((1, NN), F32),
        jax.ShapeDtypeStruct((ER, EC), F32),
        jax.ShapeDtypeStruct((ER, EC), F32),
    ],
)


def _leaky(v, sl):
    return jnp.where(v > 0, v, sl * v)


def _tc_comb_body(degp, aes1p, aes2p, den1p, deg, aes1, aes2, den1):
    deg[...] = jnp.sum(degp[...], axis=0, keepdims=True)
    aes1[...] = jnp.sum(aes1p[...], axis=0, keepdims=True)
    aes2[...] = jnp.sum(aes2p[...], axis=0, keepdims=True)
    den1[...] = jnp.sum(den1p[...], axis=0, keepdims=True)


_tc_comb = pl.pallas_call(
    _tc_comb_body,
    out_shape=[jax.ShapeDtypeStruct((1, NN), F32)] * 4,
)


def _tc_mid_body(degR, aes1R, aes2R, den1R, as1, ad1, num1T, h1T,
                 b1, gamma, beta, W2T, atts2, attd2,
                 h2T, as2, ad2, exl2):
    deg = degR[...]
    aes1 = aes1R[...]
    aes2 = aes2R[...]
    den1 = den1R[...]
    ael1 = jnp.where(deg > 0, aes1 / jnp.maximum(deg, 1.0), 0.0)
    al1 = as1[...] + ad1[...] + ael1
    al1 = _leaky(al1, 0.2)
    exl1 = jnp.exp(jnp.minimum(al1, 60.0))
    denom1 = den1 + exl1
    gat1 = (num1T[...] + exl1 * h1T[...]) / (denom1 + 1e-16) + b1[...]
    v = gat1[:, :N]
    mu = jnp.mean(v, axis=1, keepdims=True)
    var = jnp.mean((v - mu) ** 2, axis=1, keepdims=True)
    hbn = gamma[...] * (gat1 - mu) / jnp.sqrt(var + 1e-5) + beta[...]
    hbn = _leaky(hbn, 0.01)
    h2 = jnp.dot(W2T[...], hbn, preferred_element_type=F32)
    h2T[...] = h2
    a_s = jnp.sum(h2 * atts2[...], axis=0, keepdims=True)
    a_d = jnp.sum(h2 * attd2[...], axis=0, keepdims=True)
    as2[...] = a_s
    ad2[...] = a_d
    ael2 = jnp.where(deg > 0, aes2 / jnp.maximum(deg, 1.0), 0.0)
    al2 = _leaky(a_s + a_d + ael2, 0.2)
    exl2[...] = jnp.exp(jnp.minimum(al2, 60.0))


_tc_mid = pl.pallas_call(
    _tc_mid_body,
    out_shape=[
        jax.ShapeDtypeStruct((HID, NN), F32),
        jax.ShapeDtypeStruct((1, NN), F32),
        jax.ShapeDtypeStruct((1, NN), F32),
        jax.ShapeDtypeStruct((1, NN), F32),
    ],
)


def _tc_fin_body(num2T, h2T, den2p, exl2, maskP, b2, Wf1T, bf1, Wf2T, bf2,
                 out):
    den2 = jnp.sum(den2p[...], axis=0, keepdims=True) + exl2[...]
    gat2 = (num2T[...] + exl2[...] * h2T[...]) / (den2 + 1e-16) + b2[...]
    hL = _leaky(gat2, 0.01)
    m1 = _leaky(jnp.dot(Wf1T[...], hL, preferred_element_type=F32) + bf1[...],
                0.01)
    m2 = jnp.dot(Wf2T[...], m1, preferred_element_type=F32) + bf2[...]
    out[...] = m2 * maskP[...]


_tc_fin = pl.pallas_call(
    _tc_fin_body,
    out_shape=jax.ShapeDtypeStruct((1, NN), F32),
)


def kernel(x, edge_index, edge_attr, mask, W1, att_s1, att_d1, We1, att_e1,
           b1, gamma, beta, W2, att_s2, att_d2, We2, att_e2, b2,
           Wf1, bf1, Wf2, bf2):
    src = edge_index[0].astype(I32)
    dst = edge_index[1].astype(I32)
    srcP = jnp.pad(src, (0, EP - E))
    dstP = jnp.pad(dst, (0, EP - E), constant_values=N)
    xT = jnp.pad(x.T.astype(F32), ((0, 0), (0, NN - N)))
    ea0 = jnp.pad(edge_attr[:, 0].astype(F32), (0, EP - E)).reshape(ER, EC)
    ea1 = jnp.pad(edge_attr[:, 1].astype(F32), (0, EP - E)).reshape(ER, EC)
    maskP = jnp.pad(mask.astype(F32), (0, NN - N)).reshape(1, NN)
    vv = jnp.stack([We1 @ att_e1, We2 @ att_e2]).astype(F32)

    h1T, as1, ad1, ae1, ae2 = _tc_pre(
        xT, ea0, ea1, W1.T.astype(F32),
        att_s1.reshape(HID, 1), att_d1.reshape(HID, 1), vv)

    ae1f = ae1.reshape(EP)
    ae2f = ae2.reshape(EP)
    degp, aes1p, aes2p, p1 = _sc_m1(dstP, srcP, ae1f, ae2f, as1.reshape(NN))
    ex1, den1p = _sc_m2(dstP, p1, ad1.reshape(NN))
    (num1T,) = _sc_m3(srcP, dstP, ex1, h1T)

    degC, aes1C, aes2C, den1C = _tc_comb(degp, aes1p, aes2p, den1p)
    h2T, as2, ad2, exl2 = _tc_mid(
        degC, aes1C, aes2C, den1C, as1, ad1, num1T, h1T,
        b1.reshape(HID, 1), gamma.reshape(HID, 1), beta.reshape(HID, 1),
        W2.T.astype(F32), att_s2.reshape(HID, 1), att_d2.reshape(HID, 1))

    _p2, ex2, den2p = _sc_m4(srcP, dstP, ae2f, as2.reshape(NN),
                             ad2.reshape(NN))
    (num2T,) = _sc_m3(srcP, dstP, ex2, h2T)

    out2d = _tc_fin(num2T, h2T, den2p, exl2, maskP,
                    b2.reshape(HID, 1), Wf1.T.astype(F32),
                    bf1.reshape(32, 1), Wf2.T.astype(F32),
                    bf2.reshape(1, 1))
    return out2d[0, :N]
